# Initial kernel scaffold; baseline (speedup 1.0000x reference)
import jax
import jax.numpy as jnp
from jax.experimental import pallas as pl


def kernel(api_id, status_id, node_id, depth, pos, lat, ctx, edge_index, params):
    raise NotImplementedError("write your pallas kernel here")


# speedup vs baseline: 9.2642x; 9.2642x over previous
import functools
import jax
import jax.numpy as jnp
from jax import lax
from jax.experimental import pallas as pl
from jax.experimental.pallas import tpu as pltpu
from jax.experimental.pallas import tpu_sc as plsc

NN = 50000
NP = 53248
PW = 1664
NB = PW // 128
NIDR = NP // 128
EE = 800000
EP = 802816
ER = EP // 128
ERW = ER // 32
ERT = ER // 16
TILE_N = NP // 16
TRASH = NP
GCF = 64
BLK = 2000
NBLK = NN // BLK


def _mesh():
    return plsc.VectorSubcoreMesh(core_axis_name="c", subcore_axis_name="s")


def _sc1_body(src2d, dst2d, i_api, i_sta, i_nod, i_dep, i_pos, nid1d,
              t_api, t_sta, t_nod, t_dep, t_pos, ones_in, zeros1,
              e_api, e_sta, e_nod, e_dep, e_pos, cnt_out, ploc_out, locc_out,
              idxv, rows, onesv, zb1, idsv, lastv, plocv, last_sm, sem, cnt_sp):
    c = lax.axis_index("c")
    s = lax.axis_index("s")
    wid = c * 16 + s

    pltpu.sync_copy(zeros1, zb1)

    def zloop(k, _):
        pltpu.sync_copy(zb1, cnt_sp.at[pl.ds(s * TILE_N + k * 128, 128)])
        return 0
    lax.fori_loop(0, TILE_N // 128, zloop, 0)

    for tab, ids, out in ((t_api, i_api, e_api), (t_sta, i_sta, e_sta),
                          (t_nod, i_nod, e_nod), (t_dep, i_dep, e_dep),
                          (t_pos, i_pos, e_pos)):
        def eblk(j, _, tab=tab, ids=ids, out=out):
            row = wid * NB + j
            pltpu.sync_copy(ids.at[row], idxv)
            pltpu.async_copy(tab.at[idxv], rows, sem).wait()
            pltpu.sync_copy(rows, out.at[pl.ds(row * 128, 128)])
            return 0
        lax.fori_loop(0, NB, eblk, 0)

    base = wid * PW
    pltpu.sync_copy(nid1d.at[pl.ds(base, PW)], idsv)

    def linit(j, _):
        last_sm[j] = jnp.int32(-1)
        return 0
    lax.fori_loop(0, 1024, linit, 0)

    lane_iota = lax.iota(jnp.int32, 16)

    def scan(g, _):
        d = pl.ds(g * 16, 16)
        vv = idsv[d]
        gbase = base + g * 16
        pv = jnp.full((16,), -1, jnp.int32)
        for lane in range(16):
            v = vv[lane]
            prev = last_sm[v]
            last_sm[v] = gbase + lane
            pv = jnp.where(lane_iota == lane, prev, pv)
        plocv[d] = pv
        return 0
    lax.fori_loop(0, PW // 16, scan, 0)
    pltpu.sync_copy(plocv, ploc_out.at[pl.ds(base, PW)])

    def l2v(g, _):
        pv = jnp.full((16,), -1, jnp.int32)
        for lane in range(16):
            pv = jnp.where(lane_iota == lane, last_sm[g * 16 + lane], pv)
        lastv[pl.ds(g * 16, 16)] = pv
        return 0
    lax.fori_loop(0, 64, l2v, 0)
    pltpu.sync_copy(lastv, locc_out.at[wid])

    plsc.subcore_barrier()
    pltpu.sync_copy(ones_in, onesv)

    def cblk(j, _):
        row = wid * ERW + j
        pltpu.sync_copy(src2d.at[row], idxv)
        pltpu.sync_copy(onesv, cnt_sp.at[idxv], add=True)
        pltpu.sync_copy(dst2d.at[row], idxv)
        pltpu.sync_copy(onesv, cnt_sp.at[idxv], add=True)
        return 0
    lax.fori_loop(0, ERW, cblk, 0)

    plsc.subcore_barrier()
    pltpu.sync_copy(cnt_sp.at[pl.ds(s * TILE_N, TILE_N)],
                    cnt_out.at[c, pl.ds(s * TILE_N, TILE_N)])


def _sc1(src2d, dst2d, ids5, nid1d, tabs5, ones_in, zeros1):
    f32, i32 = jnp.float32, jnp.int32
    out_type = ([jax.ShapeDtypeStruct((NP, 32), f32)] * 5
                + [jax.ShapeDtypeStruct((2, NP), f32),
                   jax.ShapeDtypeStruct((NP,), i32),
                   jax.ShapeDtypeStruct((32, 1024), i32)])
    fn = pl.kernel(
        _sc1_body, out_type=out_type, mesh=_mesh(),
        compiler_params=pltpu.CompilerParams(use_tc_tiling_on_sc=False),
        scratch_types=[
            pltpu.VMEM((128,), i32),
            pltpu.VMEM((128, 32), f32),
            pltpu.VMEM((128,), f32),
            pltpu.VMEM((128,), f32),
            pltpu.VMEM((PW,), i32),
            pltpu.VMEM((1024,), i32),
            pltpu.VMEM((PW,), i32),
            pltpu.SMEM((1024,), i32),
            pltpu.SemaphoreType.DMA,
            pltpu.VMEM_SHARED((NP,), f32),
        ])
    return fn(src2d, dst2d, *ids5, nid1d, *tabs5, ones_in, zeros1)


def _sc2_body(locc, nid1d, ploc, arange1d,
              pfin, degh, car_out, lg_out,
              rowv, carv, lgv, idsv, plv, pfv, dhv, idxb, cvals, lvals, arv, sem):
    c = lax.axis_index("c")
    s = lax.axis_index("s")
    wid = c * 16 + s
    base = wid * PW

    def vinit(j, _):
        carv[pl.ds(j * 16, 16)] = jnp.full((16,), -1, jnp.int32)
        lgv[pl.ds(j * 16, 16)] = jnp.full((16,), -1, jnp.int32)
        return 0
    lax.fori_loop(0, 64, vinit, 0)

    def rloop(j, _):
        pltpu.sync_copy(locc.at[j], rowv)
        use = (j < wid)

        def vmax(g, _):
            d = pl.ds(g * 16, 16)
            rv = rowv[d]
            lgv[d] = jnp.maximum(lgv[d], rv)
            carv[d] = jnp.where(use, jnp.maximum(carv[d], rv), carv[d])
            return 0
        lax.fori_loop(0, 64, vmax, 0)
        return 0
    lax.fori_loop(0, 32, rloop, 0)

    woff = wid * 1024
    pltpu.sync_copy(carv, car_out.at[pl.ds(woff, 1024)])
    pltpu.sync_copy(lgv, lg_out.at[pl.ds(woff, 1024)])
    pltpu.sync_copy(nid1d.at[pl.ds(base, PW)], idsv)
    pltpu.sync_copy(ploc.at[pl.ds(base, PW)], plv)

    pltpu.sync_copy(arange1d.at[pl.ds(base, PW)], arv)
    for j in range(NB):
        def mkidx(g, _, j=j):
            idxb[pl.ds(g * 16, 16)] = idsv[pl.ds(j * 128 + g * 16, 16)] + woff
            return 0
        lax.fori_loop(0, 8, mkidx, 0)
        pltpu.async_copy(car_out.at[idxb], cvals, sem).wait()
        pltpu.async_copy(lg_out.at[idxb], lvals, sem).wait()

        def grp(g, _, j=j):
            d16 = pl.ds(g * 16, 16)
            dall = pl.ds(j * 128 + g * 16, 16)
            pv = plv[dall]
            pf = jnp.where(pv >= 0, pv, cvals[d16])
            pex = jnp.where(pf >= 0, 1, 0)
            nex = jnp.where(lvals[d16] != arv[dall], 1, 0)
            pfv[dall] = pf
            dhv[dall] = (1 + pex + nex).astype(jnp.float32)
            return 0
        lax.fori_loop(0, 8, grp, 0)

    pltpu.sync_copy(pfv, pfin.at[pl.ds(base, PW)])
    pltpu.sync_copy(dhv, degh.at[pl.ds(base, PW)])


def _sc2(locc, nid1d, ploc, arange1d):
    f32, i32 = jnp.float32, jnp.int32
    fn = pl.kernel(
        _sc2_body,
        out_type=[jax.ShapeDtypeStruct((NP,), i32),
                  jax.ShapeDtypeStruct((NP,), f32),
                  jax.ShapeDtypeStruct((32 * 1024,), i32),
                  jax.ShapeDtypeStruct((32 * 1024,), i32)],
        mesh=_mesh(),
        compiler_params=pltpu.CompilerParams(use_tc_tiling_on_sc=False),
        scratch_types=[
            pltpu.VMEM((1024,), i32),
            pltpu.VMEM((1024,), i32),
            pltpu.VMEM((1024,), i32),
            pltpu.VMEM((PW,), i32),
            pltpu.VMEM((PW,), i32),
            pltpu.VMEM((PW,), i32),
            pltpu.VMEM((PW,), f32),
            pltpu.VMEM((128,), i32),
            pltpu.VMEM((128,), i32),
            pltpu.VMEM((128,), i32),
            pltpu.VMEM((PW,), i32),
            pltpu.SemaphoreType.DMA,
        ])
    return fn(locc, nid1d, ploc, arange1d)


def _sc3_body(src2d, dst2d, zlo_t, zhi_t, r_t, zeros2, zeros1,
              agg_lo, agg_hi, asig0, asig1,
              sidx, didx, rows, rvals, zb2, zb1, sem, agg_sp, asig_sp):
    c = lax.axis_index("c")
    s = lax.axis_index("s")

    pltpu.sync_copy(zeros2, zb2)
    pltpu.sync_copy(zeros1, zb1)

    def zloop(k, _):
        pltpu.sync_copy(zb2, agg_sp.at[pl.ds(s * TILE_N + k * 128, 128)])
        pltpu.sync_copy(zb1, asig_sp.at[pl.ds(s * TILE_N + k * 128, 128)])
        return 0
    lax.fori_loop(0, TILE_N // 128, zloop, 0)
    plsc.subcore_barrier()

    def edge_pass(ztab, parity):
        def eblk(j, _):
            row = s * ERT + j
            pltpu.sync_copy(src2d.at[row], sidx)
            pltpu.sync_copy(dst2d.at[row], didx)
            pltpu.async_copy(ztab.at[sidx], rows, sem).wait()
            pltpu.sync_copy(rows, agg_sp.at[didx], add=True)
            pltpu.async_copy(ztab.at[didx], rows, sem).wait()
            pltpu.sync_copy(rows, agg_sp.at[sidx], add=True)

            @pl.when(j % 2 == parity)
            def _():
                pltpu.async_copy(r_t.at[didx], rvals, sem).wait()
                pltpu.sync_copy(rvals, asig_sp.at[sidx], add=True)
                pltpu.async_copy(r_t.at[sidx], rvals, sem).wait()
                pltpu.sync_copy(rvals, asig_sp.at[didx], add=True)
            return 0
        lax.fori_loop(0, ERT, eblk, 0)

    @pl.when(c == 0)
    def _():
        edge_pass(zlo_t, 0)

    @pl.when(c == 1)
    def _():
        edge_pass(zhi_t, 1)

    plsc.subcore_barrier()
    d = pl.ds(s * TILE_N, TILE_N)

    @pl.when(c == 0)
    def _():
        pltpu.sync_copy(agg_sp.at[d], agg_lo.at[d])
        pltpu.sync_copy(asig_sp.at[d], asig0.at[d])

    @pl.when(c == 1)
    def _():
        pltpu.sync_copy(agg_sp.at[d], agg_hi.at[d])
        pltpu.sync_copy(asig_sp.at[d], asig1.at[d])


def _sc3(src2d, dst2d, zlo, zhi, r_t, zeros2, zeros1):
    f32, i32 = jnp.float32, jnp.int32
    fn = pl.kernel(
        _sc3_body,
        out_type=[jax.ShapeDtypeStruct((NP, 32), f32),
                  jax.ShapeDtypeStruct((NP, 32), f32),
                  jax.ShapeDtypeStruct((NP,), f32),
                  jax.ShapeDtypeStruct((NP,), f32)],
        mesh=_mesh(),
        compiler_params=pltpu.CompilerParams(use_tc_tiling_on_sc=False),
        scratch_types=[
            pltpu.VMEM((128,), i32),
            pltpu.VMEM((128,), i32),
            pltpu.VMEM((128, 32), f32),
            pltpu.VMEM((128,), f32),
            pltpu.VMEM((128, 32), f32),
            pltpu.VMEM((128,), f32),
            pltpu.SemaphoreType.DMA,
            pltpu.VMEM_SHARED((NP, 32), f32),
            pltpu.VMEM_SHARED((NP,), f32),
        ])
    return fn(src2d, dst2d, zlo, zhi, r_t, zeros2, zeros1)


def _sc4_body(pf2d, zhlo_t, zhhi_t, rh_t, zeros2, zeros1,
              agh_lo, agh_hi, ash0, ash1,
              pv, gp, si, sp_, rows, rvals, zb2, zb1, sem, agg_sp, asig_sp):
    c = lax.axis_index("c")
    s = lax.axis_index("s")
    wid = c * 16 + s

    pltpu.sync_copy(zeros2, zb2)
    pltpu.sync_copy(zeros1, zb1)

    def zloop(k, _):
        pltpu.sync_copy(zb2, agg_sp.at[pl.ds(s * TILE_N + k * 128, 128)])
        pltpu.sync_copy(zb1, asig_sp.at[pl.ds(s * TILE_N + k * 128, 128)])
        return 0
    lax.fori_loop(0, TILE_N // 128, zloop, 0)
    plsc.subcore_barrier()

    def host_pass(ztab, parity):
        def blk(j, _):
            row = s * (2 * NB) + j
            nodebase = row * 128
            pltpu.sync_copy(pf2d.at[row], pv)

            def grp(g, _):
                d = pl.ds(g * 16, 16)
                pvv = pv[d]
                ivv = nodebase + g * 16 + lax.iota(jnp.int32, 16)
                m = pvv >= 0
                gp[d] = jnp.where(m, pvv, 0)
                si[d] = jnp.where(m, ivv, TRASH)
                sp_[d] = jnp.where(m, pvv, TRASH)
                return 0
            lax.fori_loop(0, 8, grp, 0)

            pltpu.async_copy(ztab.at[gp], rows, sem).wait()
            pltpu.sync_copy(rows, agg_sp.at[si], add=True)
            pltpu.sync_copy(ztab.at[pl.ds(nodebase, 128)], rows)
            pltpu.sync_copy(rows, agg_sp.at[sp_], add=True)

            @pl.when(j % 2 == parity)
            def _():
                pltpu.async_copy(rh_t.at[gp], rvals, sem).wait()
                pltpu.sync_copy(rvals, asig_sp.at[si], add=True)
                pltpu.sync_copy(rh_t.at[pl.ds(nodebase, 128)], rvals)
                pltpu.sync_copy(rvals, asig_sp.at[sp_], add=True)
            return 0
        lax.fori_loop(0, 2 * NB, blk, 0)

    @pl.when(c == 0)
    def _():
        host_pass(zhlo_t, 0)

    @pl.when(c == 1)
    def _():
        host_pass(zhhi_t, 1)

    plsc.subcore_barrier()
    d = pl.ds(s * TILE_N, TILE_N)

    @pl.when(c == 0)
    def _():
        pltpu.sync_copy(agg_sp.at[d], agh_lo.at[d])
        pltpu.sync_copy(asig_sp.at[d], ash0.at[d])

    @pl.when(c == 1)
    def _():
        pltpu.sync_copy(agg_sp.at[d], agh_hi.at[d])
        pltpu.sync_copy(asig_sp.at[d], ash1.at[d])


def _sc4(pf2d, zhlo, zhhi, rh_t, zeros2, zeros1):
    f32, i32 = jnp.float32, jnp.int32
    fn = pl.kernel(
        _sc4_body,
        out_type=[jax.ShapeDtypeStruct((NP, 32), f32),
                  jax.ShapeDtypeStruct((NP, 32), f32),
                  jax.ShapeDtypeStruct((NP,), f32),
                  jax.ShapeDtypeStruct((NP,), f32)],
        mesh=_mesh(),
        compiler_params=pltpu.CompilerParams(use_tc_tiling_on_sc=False),
        scratch_types=[
            pltpu.VMEM((128,), i32),
            pltpu.VMEM((128,), i32),
            pltpu.VMEM((128,), i32),
            pltpu.VMEM((128,), i32),
            pltpu.VMEM((128, 32), f32),
            pltpu.VMEM((128,), f32),
            pltpu.VMEM((128, 32), f32),
            pltpu.VMEM((128,), f32),
            pltpu.SemaphoreType.DMA,
            pltpu.VMEM_SHARED((TRASH + 8, 32), f32),
            pltpu.VMEM_SHARED((TRASH + 8,), f32),
        ])
    return fn(pf2d, zhlo, zhhi, rh_t, zeros2, zeros1)


def _tca_body(eapi, esta, enod, edep, epos, lat, ctx, cnt0, cnt1, degh,
              lw1, lb1, lw2, lb2, mw, mb, g1w, h1w, wiou, biou,
              zlo, zhi, zhlo, zhhi, r_o, rh_o, shr_o, sctx_o):
    i = pl.program_id(0)
    lt = jnp.maximum(lat[...] @ lw1[...] + lb1[...], 0.0) @ lw2[...] + lb2[...]
    feat = jnp.concatenate(
        [eapi[...], esta[...], enod[...], edep[...], epos[...], lt], axis=1)
    x = jnp.maximum(feat @ mw[...] + mb[...], 0.0)

    r = lax.rsqrt(cnt0[...] + cnt1[...] + 1.0)
    rh = lax.rsqrt(degh[...])
    z = (x * r) @ g1w[...]
    zh = (x * rh) @ h1w[...]
    zlo[...] = z[:, :32]
    zhi[...] = z[:, 32:]
    zhlo[...] = zh[:, :32]
    zhhi[...] = zh[:, 32:]
    r_o[...] = r
    rh_o[...] = rh

    iou = x @ wiou[...] + biou[...]
    ig = jax.nn.sigmoid(iou[:, :GCF])
    og = jax.nn.sigmoid(iou[:, GCF:2 * GCF])
    ug = jnp.tanh(iou[:, 2 * GCF:])
    h = og * jnp.tanh(ig * ug)
    hr = jnp.maximum(h, 0.0)

    @pl.when(i == 0)
    def _():
        shr_o[...] = jnp.zeros_like(shr_o)
        sctx_o[...] = jnp.zeros_like(sctx_o)

    shr_o[...] += jnp.sum(hr, axis=0, keepdims=True)
    sctx_o[...] += jnp.sum(ctx[...], axis=0, keepdims=True)


def _tca(embs, lat, ctx, cnt0, cnt1, degh, p):
    f32 = jnp.float32
    node = lambda w: pl.BlockSpec((BLK, w), lambda i: (i, 0))
    full = lambda a: pl.BlockSpec(a.shape, lambda i: (0,) * a.ndim)
    acc = lambda w: pl.BlockSpec((1, w), lambda i: (0, 0))
    wargs = (p['lat_w1'], p['lat_b1'].reshape(1, 32), p['lat_w2'],
             p['lat_b2'].reshape(1, 32), p['merge_w'],
             p['merge_b'].reshape(1, 64), p['gcn1_w'], p['hgc1_w'],
             p['W_iouf'][:, :3 * GCF], p['b_iou'])
    fn = pl.pallas_call(
        _tca_body,
        grid=(NBLK,),
        in_specs=[node(32)] * 5 + [node(1), node(7), node(1), node(1), node(1)]
                 + [full(w) for w in wargs],
        out_specs=[node(32)] * 4 + [node(1)] * 2 + [acc(64), acc(7)],
        out_shape=[jax.ShapeDtypeStruct((NP, 32), f32)] * 4
                  + [jax.ShapeDtypeStruct((NP, 1), f32)] * 2
                  + [jax.ShapeDtypeStruct((1, 64), f32),
                     jax.ShapeDtypeStruct((1, 7), f32)],
    )
    return fn(*embs, lat, ctx, cnt0, cnt1, degh, *wargs)


def _tcb_body(zlo, zhi, zhlo, zhhi, alo, ahi, hlo, hhi,
              as0, as1, ah0, ah1, r_i, rh_i, shr, sctx,
              g1b, h1b, g2w, g2b, hg2w, hg2b, tow, tob, cw, cb,
              fw, fb, bw, bb, c3w, c3b, tw, tb,
              ob, oc3, oty, acc_c, acc_h):
    i = pl.program_id(0)

    @pl.when(i == 0)
    def _():
        acc_c[...] = jnp.zeros_like(acc_c)
        acc_h[...] = jnp.zeros_like(acc_h)

    r = r_i[...]
    z = jnp.concatenate([zlo[...], zhi[...]], axis=1)
    agg = jnp.concatenate([alo[...], ahi[...]], axis=1)
    h1 = jnp.maximum(r * (agg + z) + g1b[...], 0.0)
    alpha = r * (as0[...] + as1[...] + r)
    acc_c[...] += jnp.sum(alpha * h1, axis=0, keepdims=True)

    rh = rh_i[...]
    zh = jnp.concatenate([zhlo[...], zhhi[...]], axis=1)
    aggh = jnp.concatenate([hlo[...], hhi[...]], axis=1)
    h1h = jnp.maximum(rh * (aggh + zh) + h1b[...], 0.0)
    alphah = rh * (ah0[...] + ah1[...] + rh)
    acc_h[...] += jnp.sum(alphah * h1h, axis=0, keepdims=True)

    @pl.when(i == NBLK - 1)
    def _():
        inv = 1.0 / NN
        mc = (acc_c[...] * inv) @ g2w[...] + g2b[...]
        mh = (acc_h[...] * inv) @ hg2w[...] + hg2b[...]
        mt = (shr[...] * inv) @ tow[...] + tob[...]
        ch = jnp.maximum((sctx[...] * inv) @ cw[...] + cb[...], 0.0)
        fused = jnp.maximum(
            jnp.concatenate([mc, mh, mt, ch], axis=1) @ fw[...] + fb[...], 0.0)
        ob[...] = fused @ bw[...] + bb[...]
        oc3[...] = fused @ c3w[...] + c3b[...]
        oty[...] = fused @ tw[...] + tb[...]


def _tcb(zs, aggs, asigs, r_t, rh_t, shr, sctx, p):
    f32 = jnp.float32
    node = lambda w: pl.BlockSpec((BLK, w), lambda i: (i, 0))
    full = lambda a: pl.BlockSpec(a.shape, lambda i: (0,) * a.ndim)
    wargs = (p['gcn1_b'].reshape(1, 64), p['hgc1_b'].reshape(1, 64),
             p['gcn2_w'], p['gcn2_b'].reshape(1, 64),
             p['hgc2_w'], p['hgc2_b'].reshape(1, 64),
             p['tl_out_w'], p['tl_out_b'].reshape(1, 64),
             p['ctx_w'], p['ctx_b'].reshape(1, 64),
             p['fuse_w'], p['fuse_b'].reshape(1, 128),
             p['bin_w'], p['bin_b'].reshape(1, 1),
             p['c3_w'], p['c3_b'].reshape(1, 3),
             p['type_w'], p['type_b'].reshape(1, 12))
    fn = pl.pallas_call(
        _tcb_body,
        grid=(NBLK,),
        in_specs=[node(32)] * 8 + [node(1)] * 6
                 + [pl.BlockSpec((1, 64), lambda i: (0, 0)),
                    pl.BlockSpec((1, 7), lambda i: (0, 0))]
                 + [full(w) for w in wargs],
        out_specs=[pl.BlockSpec((1, 1), lambda i: (0, 0)),
                   pl.BlockSpec((1, 3), lambda i: (0, 0)),
                   pl.BlockSpec((1, 12), lambda i: (0, 0))],
        out_shape=[jax.ShapeDtypeStruct((1, 1), f32),
                   jax.ShapeDtypeStruct((1, 3), f32),
                   jax.ShapeDtypeStruct((1, 12), f32)],
        scratch_shapes=[pltpu.VMEM((1, 64), f32), pltpu.VMEM((1, 64), f32)],
    )
    return fn(*zs, *aggs, *asigs, r_t, rh_t, shr, sctx, *wargs)


@jax.jit
def kernel(api_id, status_id, node_id, depth, pos, lat, ctx, edge_index, params):
    p = params
    f32, i32 = jnp.float32, jnp.int32

    def pad2d(a):
        return jnp.pad(a.astype(i32), (0, NP - NN)).reshape(NIDR, 128)

    ids5 = (pad2d(api_id), pad2d(status_id), pad2d(node_id),
            pad2d(depth), pad2d(pos))
    nid1d = jnp.pad(node_id.astype(i32), (0, NP - NN), constant_values=1023)
    src2d = jnp.pad(edge_index[0].astype(i32), (0, EP - EE),
                    constant_values=NN).reshape(ER, 128)
    dst2d = jnp.pad(edge_index[1].astype(i32), (0, EP - EE),
                    constant_values=NN).reshape(ER, 128)
    ones_in = jnp.ones((128,), f32)
    zeros1 = jnp.zeros((128,), f32)
    zeros2 = jnp.zeros((128, 32), f32)
    tabs5 = (p['api_emb'], p['status_emb'], p['node_emb'],
             p['depth_emb'], p['pos_emb'])

    *embs, cnt, ploc, locc = _sc1(src2d, dst2d, ids5, nid1d, tabs5,
                                  ones_in, zeros1)
    pfin, degh, _, _ = _sc2(locc, nid1d, ploc, jnp.arange(NP, dtype=i32))

    zlo, zhi, zhlo, zhhi, r_t, rh_t, shr, sctx = _tca(
        embs, lat, ctx, cnt[0].reshape(NP, 1), cnt[1].reshape(NP, 1),
        degh.reshape(NP, 1), p)

    agg_lo, agg_hi, as0, as1 = _sc3(src2d, dst2d, zlo, zhi, r_t.reshape(NP),
                                    zeros2, zeros1)
    agh_lo, agh_hi, ah0, ah1 = _sc4(pfin.reshape(NIDR, 128), zhlo, zhhi,
                                    rh_t.reshape(NP), zeros2, zeros1)

    ob, oc3, oty = _tcb((zlo, zhi, zhlo, zhhi),
                        (agg_lo, agg_hi, agh_lo, agh_hi),
                        (as0.reshape(NP, 1), as1.reshape(NP, 1),
                         ah0.reshape(NP, 1), ah1.reshape(NP, 1)),
                        r_t, rh_t, shr, sctx, p)
    return ob.reshape(1), oc3, oty


# speedup vs baseline: 11.0760x; 1.1956x over previous
import functools
import jax
import jax.numpy as jnp
from jax import lax
from jax.experimental import pallas as pl
from jax.experimental.pallas import tpu as pltpu
from jax.experimental.pallas import tpu_sc as plsc

NN = 50000
NP = 53248
PW = 1664
NB = PW // 128
NIDR = NP // 128
EE = 800000
EP = 802816
ER = EP // 128
ERW = ER // 32
ERT = ER // 16
TILE_N = NP // 16
TRASH = NP
GCF = 64
BLK = 2000
NBLK = NN // BLK


def _mesh():
    return plsc.VectorSubcoreMesh(core_axis_name="c", subcore_axis_name="s")


def _sc1_body(src2d, dst2d, i_api, i_sta, i_nod, i_dep, i_pos, nid1d,
              t_api, t_sta, t_nod, t_dep, t_pos, ones_in, zeros1,
              e_api, e_sta, e_nod, e_dep, e_pos, cnt_out, ploc_out, locc_out,
              idxv, rows, onesv, zb1, idsv, lastv, plocv, last_sm, sem, cnt_sp):
    c = lax.axis_index("c")
    s = lax.axis_index("s")
    wid = c * 16 + s

    pltpu.sync_copy(zeros1, zb1)

    def zloop(k, _):
        pltpu.sync_copy(zb1, cnt_sp.at[pl.ds(s * TILE_N + k * 128, 128)])
        return 0
    lax.fori_loop(0, TILE_N // 128, zloop, 0)

    for tab, ids, out in ((t_api, i_api, e_api), (t_sta, i_sta, e_sta),
                          (t_nod, i_nod, e_nod), (t_dep, i_dep, e_dep),
                          (t_pos, i_pos, e_pos)):
        def eblk(j, _, tab=tab, ids=ids, out=out):
            row = wid * NB + j
            pltpu.sync_copy(ids.at[row], idxv)
            pltpu.async_copy(tab.at[idxv], rows, sem).wait()
            pltpu.sync_copy(rows, out.at[pl.ds(row * 128, 128)])
            return 0
        lax.fori_loop(0, NB, eblk, 0)

    base = wid * PW
    pltpu.sync_copy(nid1d.at[pl.ds(base, PW)], idsv)

    def linit(j, _):
        last_sm[j] = jnp.int32(-1)
        return 0
    lax.fori_loop(0, 1024, linit, 0)

    lane_iota = lax.iota(jnp.int32, 16)

    def scan(g, _):
        d = pl.ds(g * 16, 16)
        vv = idsv[d]
        gbase = base + g * 16
        pv = jnp.full((16,), -1, jnp.int32)
        for lane in range(16):
            v = vv[lane]
            prev = last_sm[v]
            last_sm[v] = gbase + lane
            pv = jnp.where(lane_iota == lane, prev, pv)
        plocv[d] = pv
        return 0
    lax.fori_loop(0, PW // 16, scan, 0)
    pltpu.sync_copy(plocv, ploc_out.at[pl.ds(base, PW)])

    def l2v(g, _):
        pv = jnp.full((16,), -1, jnp.int32)
        for lane in range(16):
            pv = jnp.where(lane_iota == lane, last_sm[g * 16 + lane], pv)
        lastv[pl.ds(g * 16, 16)] = pv
        return 0
    lax.fori_loop(0, 64, l2v, 0)
    pltpu.sync_copy(lastv, locc_out.at[wid])

    plsc.subcore_barrier()
    pltpu.sync_copy(ones_in, onesv)

    def cblk(j, _):
        row = wid * ERW + j
        pltpu.sync_copy(src2d.at[row], idxv)
        pltpu.sync_copy(onesv, cnt_sp.at[idxv], add=True)
        pltpu.sync_copy(dst2d.at[row], idxv)
        pltpu.sync_copy(onesv, cnt_sp.at[idxv], add=True)
        return 0
    lax.fori_loop(0, ERW, cblk, 0)

    plsc.subcore_barrier()
    pltpu.sync_copy(cnt_sp.at[pl.ds(s * TILE_N, TILE_N)],
                    cnt_out.at[c, pl.ds(s * TILE_N, TILE_N)])


def _sc1(src2d, dst2d, ids5, nid1d, tabs5, ones_in, zeros1):
    f32, i32 = jnp.float32, jnp.int32
    out_type = ([jax.ShapeDtypeStruct((NP, 32), f32)] * 5
                + [jax.ShapeDtypeStruct((2, NP), f32),
                   jax.ShapeDtypeStruct((NP,), i32),
                   jax.ShapeDtypeStruct((32, 1024), i32)])
    fn = pl.kernel(
        _sc1_body, out_type=out_type, mesh=_mesh(),
        compiler_params=pltpu.CompilerParams(use_tc_tiling_on_sc=False),
        scratch_types=[
            pltpu.VMEM((128,), i32),
            pltpu.VMEM((128, 32), f32),
            pltpu.VMEM((128,), f32),
            pltpu.VMEM((128,), f32),
            pltpu.VMEM((PW,), i32),
            pltpu.VMEM((1024,), i32),
            pltpu.VMEM((PW,), i32),
            pltpu.SMEM((1024,), i32),
            pltpu.SemaphoreType.DMA,
            pltpu.VMEM_SHARED((NP,), f32),
        ])
    return fn(src2d, dst2d, *ids5, nid1d, *tabs5, ones_in, zeros1)


def _sc2_body(locc, nid1d, ploc, arange1d,
              pfin, degh, car_out, lg_out,
              rowv, carv, lgv, idsv, plv, pfv, dhv, idxb, cvals, lvals, arv, sem):
    c = lax.axis_index("c")
    s = lax.axis_index("s")
    wid = c * 16 + s
    base = wid * PW

    def vinit(j, _):
        carv[pl.ds(j * 16, 16)] = jnp.full((16,), -1, jnp.int32)
        lgv[pl.ds(j * 16, 16)] = jnp.full((16,), -1, jnp.int32)
        return 0
    lax.fori_loop(0, 64, vinit, 0)

    def rloop(j, _):
        pltpu.sync_copy(locc.at[j], rowv)
        use = (j < wid)

        def vmax(g, _):
            d = pl.ds(g * 16, 16)
            rv = rowv[d]
            lgv[d] = jnp.maximum(lgv[d], rv)
            carv[d] = jnp.where(use, jnp.maximum(carv[d], rv), carv[d])
            return 0
        lax.fori_loop(0, 64, vmax, 0)
        return 0
    lax.fori_loop(0, 32, rloop, 0)

    woff = wid * 1024
    pltpu.sync_copy(carv, car_out.at[pl.ds(woff, 1024)])
    pltpu.sync_copy(lgv, lg_out.at[pl.ds(woff, 1024)])
    pltpu.sync_copy(nid1d.at[pl.ds(base, PW)], idsv)
    pltpu.sync_copy(ploc.at[pl.ds(base, PW)], plv)

    pltpu.sync_copy(arange1d.at[pl.ds(base, PW)], arv)
    for j in range(NB):
        def mkidx(g, _, j=j):
            idxb[pl.ds(g * 16, 16)] = idsv[pl.ds(j * 128 + g * 16, 16)] + woff
            return 0
        lax.fori_loop(0, 8, mkidx, 0)
        pltpu.async_copy(car_out.at[idxb], cvals, sem).wait()
        pltpu.async_copy(lg_out.at[idxb], lvals, sem).wait()

        def grp(g, _, j=j):
            d16 = pl.ds(g * 16, 16)
            dall = pl.ds(j * 128 + g * 16, 16)
            pv = plv[dall]
            pf = jnp.where(pv >= 0, pv, cvals[d16])
            pex = jnp.where(pf >= 0, 1, 0)
            nex = jnp.where(lvals[d16] != arv[dall], 1, 0)
            pfv[dall] = pf
            dhv[dall] = (1 + pex + nex).astype(jnp.float32)
            return 0
        lax.fori_loop(0, 8, grp, 0)

    pltpu.sync_copy(pfv, pfin.at[pl.ds(base, PW)])
    pltpu.sync_copy(dhv, degh.at[pl.ds(base, PW)])


def _sc2(locc, nid1d, ploc, arange1d):
    f32, i32 = jnp.float32, jnp.int32
    fn = pl.kernel(
        _sc2_body,
        out_type=[jax.ShapeDtypeStruct((NP,), i32),
                  jax.ShapeDtypeStruct((NP,), f32),
                  jax.ShapeDtypeStruct((32 * 1024,), i32),
                  jax.ShapeDtypeStruct((32 * 1024,), i32)],
        mesh=_mesh(),
        compiler_params=pltpu.CompilerParams(use_tc_tiling_on_sc=False),
        scratch_types=[
            pltpu.VMEM((1024,), i32),
            pltpu.VMEM((1024,), i32),
            pltpu.VMEM((1024,), i32),
            pltpu.VMEM((PW,), i32),
            pltpu.VMEM((PW,), i32),
            pltpu.VMEM((PW,), i32),
            pltpu.VMEM((PW,), f32),
            pltpu.VMEM((128,), i32),
            pltpu.VMEM((128,), i32),
            pltpu.VMEM((128,), i32),
            pltpu.VMEM((PW,), i32),
            pltpu.SemaphoreType.DMA,
        ])
    return fn(locc, nid1d, ploc, arange1d)


def _sc3_body(src2d, dst2d, zlo_t, zhi_t, r_t, zeros2, zeros1,
              agg_lo, agg_hi, asig0, asig1,
              sidxA, didxA, rowsA0, rowsA1,
              rvals0, rvals1, zb2, zb1, semZ, agg_sp, asig_sp):
    c = lax.axis_index("c")
    s = lax.axis_index("s")

    pltpu.sync_copy(zeros2, zb2)
    pltpu.sync_copy(zeros1, zb1)

    def zloop(k, _):
        pltpu.sync_copy(zb2, agg_sp.at[pl.ds(s * TILE_N + k * 128, 128)])
        pltpu.sync_copy(zb1, asig_sp.at[pl.ds(s * TILE_N + k * 128, 128)])
        return 0
    lax.fori_loop(0, TILE_N // 128, zloop, 0)
    plsc.subcore_barrier()

    def edge_pass(ztab, parity):
        def eblk(j, _):
            row = s * ERT + j
            pltpu.sync_copy(src2d.at[row], sidxA)
            pltpu.sync_copy(dst2d.at[row], didxA)
            a0 = pltpu.async_copy(ztab.at[sidxA], rowsA0, semZ)
            a1 = pltpu.async_copy(ztab.at[didxA], rowsA1, semZ)
            do_alpha = (j % 2 == parity)

            @pl.when(do_alpha)
            def _():
                r0 = pltpu.async_copy(r_t.at[didxA], rvals0, semZ)
                r1 = pltpu.async_copy(r_t.at[sidxA], rvals1, semZ)
                r0.wait()
                r1.wait()
            a0.wait()
            a1.wait()
            pltpu.sync_copy(rowsA0, agg_sp.at[didxA], add=True)
            pltpu.sync_copy(rowsA1, agg_sp.at[sidxA], add=True)

            @pl.when(do_alpha)
            def _():
                pltpu.sync_copy(rvals0, asig_sp.at[sidxA], add=True)
                pltpu.sync_copy(rvals1, asig_sp.at[didxA], add=True)
            return 0
        lax.fori_loop(0, ERT, eblk, 0)

    @pl.when(c == 0)
    def _():
        edge_pass(zlo_t, 0)

    @pl.when(c == 1)
    def _():
        edge_pass(zhi_t, 1)

    plsc.subcore_barrier()
    d = pl.ds(s * TILE_N, TILE_N)

    @pl.when(c == 0)
    def _():
        pltpu.sync_copy(agg_sp.at[d], agg_lo.at[d])
        pltpu.sync_copy(asig_sp.at[d], asig0.at[d])

    @pl.when(c == 1)
    def _():
        pltpu.sync_copy(agg_sp.at[d], agg_hi.at[d])
        pltpu.sync_copy(asig_sp.at[d], asig1.at[d])


def _sc3(src2d, dst2d, zlo, zhi, r_t, zeros2, zeros1):
    f32, i32 = jnp.float32, jnp.int32
    fn = pl.kernel(
        _sc3_body,
        out_type=[jax.ShapeDtypeStruct((NP, 32), f32),
                  jax.ShapeDtypeStruct((NP, 32), f32),
                  jax.ShapeDtypeStruct((NP,), f32),
                  jax.ShapeDtypeStruct((NP,), f32)],
        mesh=_mesh(),
        compiler_params=pltpu.CompilerParams(use_tc_tiling_on_sc=False),
        scratch_types=[
            pltpu.VMEM((128,), i32),
            pltpu.VMEM((128,), i32),
            pltpu.VMEM((128, 32), f32),
            pltpu.VMEM((128, 32), f32),
            pltpu.VMEM((128,), f32),
            pltpu.VMEM((128,), f32),
            pltpu.VMEM((128, 32), f32),
            pltpu.VMEM((128,), f32),
            pltpu.SemaphoreType.DMA,
            pltpu.VMEM_SHARED((NP, 32), f32),
            pltpu.VMEM_SHARED((NP,), f32),
        ])
    return fn(src2d, dst2d, zlo, zhi, r_t, zeros2, zeros1)


def _sc4_body(pf2d, zhlo_t, zhhi_t, rh_t, zeros2, zeros1,
              agh_lo, agh_hi, ash0, ash1,
              pv, gp, si, sp_, rows, rvals, zb2, zb1, sem, agg_sp, asig_sp):
    c = lax.axis_index("c")
    s = lax.axis_index("s")
    wid = c * 16 + s

    pltpu.sync_copy(zeros2, zb2)
    pltpu.sync_copy(zeros1, zb1)

    def zloop(k, _):
        pltpu.sync_copy(zb2, agg_sp.at[pl.ds(s * TILE_N + k * 128, 128)])
        pltpu.sync_copy(zb1, asig_sp.at[pl.ds(s * TILE_N + k * 128, 128)])
        return 0
    lax.fori_loop(0, TILE_N // 128, zloop, 0)
    plsc.subcore_barrier()

    def host_pass(ztab, parity):
        def blk(j, _):
            row = s * (2 * NB) + j
            nodebase = row * 128
            pltpu.sync_copy(pf2d.at[row], pv)

            def grp(g, _):
                d = pl.ds(g * 16, 16)
                pvv = pv[d]
                ivv = nodebase + g * 16 + lax.iota(jnp.int32, 16)
                m = pvv >= 0
                gp[d] = jnp.where(m, pvv, 0)
                si[d] = jnp.where(m, ivv, TRASH)
                sp_[d] = jnp.where(m, pvv, TRASH)
                return 0
            lax.fori_loop(0, 8, grp, 0)

            pltpu.async_copy(ztab.at[gp], rows, sem).wait()
            pltpu.sync_copy(rows, agg_sp.at[si], add=True)
            pltpu.sync_copy(ztab.at[pl.ds(nodebase, 128)], rows)
            pltpu.sync_copy(rows, agg_sp.at[sp_], add=True)

            @pl.when(j % 2 == parity)
            def _():
                pltpu.async_copy(rh_t.at[gp], rvals, sem).wait()
                pltpu.sync_copy(rvals, asig_sp.at[si], add=True)
                pltpu.sync_copy(rh_t.at[pl.ds(nodebase, 128)], rvals)
                pltpu.sync_copy(rvals, asig_sp.at[sp_], add=True)
            return 0
        lax.fori_loop(0, 2 * NB, blk, 0)

    @pl.when(c == 0)
    def _():
        host_pass(zhlo_t, 0)

    @pl.when(c == 1)
    def _():
        host_pass(zhhi_t, 1)

    plsc.subcore_barrier()
    d = pl.ds(s * TILE_N, TILE_N)

    @pl.when(c == 0)
    def _():
        pltpu.sync_copy(agg_sp.at[d], agh_lo.at[d])
        pltpu.sync_copy(asig_sp.at[d], ash0.at[d])

    @pl.when(c == 1)
    def _():
        pltpu.sync_copy(agg_sp.at[d], agh_hi.at[d])
        pltpu.sync_copy(asig_sp.at[d], ash1.at[d])


def _sc4(pf2d, zhlo, zhhi, rh_t, zeros2, zeros1):
    f32, i32 = jnp.float32, jnp.int32
    fn = pl.kernel(
        _sc4_body,
        out_type=[jax.ShapeDtypeStruct((NP, 32), f32),
                  jax.ShapeDtypeStruct((NP, 32), f32),
                  jax.ShapeDtypeStruct((NP,), f32),
                  jax.ShapeDtypeStruct((NP,), f32)],
        mesh=_mesh(),
        compiler_params=pltpu.CompilerParams(use_tc_tiling_on_sc=False),
        scratch_types=[
            pltpu.VMEM((128,), i32),
            pltpu.VMEM((128,), i32),
            pltpu.VMEM((128,), i32),
            pltpu.VMEM((128,), i32),
            pltpu.VMEM((128, 32), f32),
            pltpu.VMEM((128,), f32),
            pltpu.VMEM((128, 32), f32),
            pltpu.VMEM((128,), f32),
            pltpu.SemaphoreType.DMA,
            pltpu.VMEM_SHARED((TRASH + 8, 32), f32),
            pltpu.VMEM_SHARED((TRASH + 8,), f32),
        ])
    return fn(pf2d, zhlo, zhhi, rh_t, zeros2, zeros1)


def _tca_body(eapi, esta, enod, edep, epos, lat, ctx, cnt0, cnt1, degh,
              lw1, lb1, lw2, lb2, mw, mb, g1w, h1w, wiou, biou,
              zlo, zhi, zhlo, zhhi, r_o, rh_o, shr_o, sctx_o):
    i = pl.program_id(0)
    lt = jnp.maximum(lat[...] @ lw1[...] + lb1[...], 0.0) @ lw2[...] + lb2[...]
    feat = jnp.concatenate(
        [eapi[...], esta[...], enod[...], edep[...], epos[...], lt], axis=1)
    x = jnp.maximum(feat @ mw[...] + mb[...], 0.0)

    r = lax.rsqrt(cnt0[...] + cnt1[...] + 1.0)
    rh = lax.rsqrt(degh[...])
    z = (x * r) @ g1w[...]
    zh = (x * rh) @ h1w[...]
    zlo[...] = z[:, :32]
    zhi[...] = z[:, 32:]
    zhlo[...] = zh[:, :32]
    zhhi[...] = zh[:, 32:]
    r_o[...] = r
    rh_o[...] = rh

    iou = x @ wiou[...] + biou[...]
    ig = jax.nn.sigmoid(iou[:, :GCF])
    og = jax.nn.sigmoid(iou[:, GCF:2 * GCF])
    ug = jnp.tanh(iou[:, 2 * GCF:])
    h = og * jnp.tanh(ig * ug)
    hr = jnp.maximum(h, 0.0)

    @pl.when(i == 0)
    def _():
        shr_o[...] = jnp.zeros_like(shr_o)
        sctx_o[...] = jnp.zeros_like(sctx_o)

    shr_o[...] += jnp.sum(hr, axis=0, keepdims=True)
    sctx_o[...] += jnp.sum(ctx[...], axis=0, keepdims=True)


def _tca(embs, lat, ctx, cnt0, cnt1, degh, p):
    f32 = jnp.float32
    node = lambda w: pl.BlockSpec((BLK, w), lambda i: (i, 0))
    full = lambda a: pl.BlockSpec(a.shape, lambda i: (0,) * a.ndim)
    acc = lambda w: pl.BlockSpec((1, w), lambda i: (0, 0))
    wargs = (p['lat_w1'], p['lat_b1'].reshape(1, 32), p['lat_w2'],
             p['lat_b2'].reshape(1, 32), p['merge_w'],
             p['merge_b'].reshape(1, 64), p['gcn1_w'], p['hgc1_w'],
             p['W_iouf'][:, :3 * GCF], p['b_iou'])
    fn = pl.pallas_call(
        _tca_body,
        grid=(NBLK,),
        in_specs=[node(32)] * 5 + [node(1), node(7), node(1), node(1), node(1)]
                 + [full(w) for w in wargs],
        out_specs=[node(32)] * 4 + [node(1)] * 2 + [acc(64), acc(7)],
        out_shape=[jax.ShapeDtypeStruct((NP, 32), f32)] * 4
                  + [jax.ShapeDtypeStruct((NP, 1), f32)] * 2
                  + [jax.ShapeDtypeStruct((1, 64), f32),
                     jax.ShapeDtypeStruct((1, 7), f32)],
    )
    return fn(*embs, lat, ctx, cnt0, cnt1, degh, *wargs)


def _tcb_body(zlo, zhi, zhlo, zhhi, alo, ahi, hlo, hhi,
              as0, as1, ah0, ah1, r_i, rh_i, shr, sctx,
              g1b, h1b, g2w, g2b, hg2w, hg2b, tow, tob, cw, cb,
              fw, fb, bw, bb, c3w, c3b, tw, tb,
              ob, oc3, oty, acc_c, acc_h):
    i = pl.program_id(0)

    @pl.when(i == 0)
    def _():
        acc_c[...] = jnp.zeros_like(acc_c)
        acc_h[...] = jnp.zeros_like(acc_h)

    r = r_i[...]
    z = jnp.concatenate([zlo[...], zhi[...]], axis=1)
    agg = jnp.concatenate([alo[...], ahi[...]], axis=1)
    h1 = jnp.maximum(r * (agg + z) + g1b[...], 0.0)
    alpha = r * (as0[...] + as1[...] + r)
    acc_c[...] += jnp.sum(alpha * h1, axis=0, keepdims=True)

    rh = rh_i[...]
    zh = jnp.concatenate([zhlo[...], zhhi[...]], axis=1)
    aggh = jnp.concatenate([hlo[...], hhi[...]], axis=1)
    h1h = jnp.maximum(rh * (aggh + zh) + h1b[...], 0.0)
    alphah = rh * (ah0[...] + ah1[...] + rh)
    acc_h[...] += jnp.sum(alphah * h1h, axis=0, keepdims=True)

    @pl.when(i == NBLK - 1)
    def _():
        inv = 1.0 / NN
        mc = (acc_c[...] * inv) @ g2w[...] + g2b[...]
        mh = (acc_h[...] * inv) @ hg2w[...] + hg2b[...]
        mt = (shr[...] * inv) @ tow[...] + tob[...]
        ch = jnp.maximum((sctx[...] * inv) @ cw[...] + cb[...], 0.0)
        fused = jnp.maximum(
            jnp.concatenate([mc, mh, mt, ch], axis=1) @ fw[...] + fb[...], 0.0)
        ob[...] = fused @ bw[...] + bb[...]
        oc3[...] = fused @ c3w[...] + c3b[...]
        oty[...] = fused @ tw[...] + tb[...]


def _tcb(zs, aggs, asigs, r_t, rh_t, shr, sctx, p):
    f32 = jnp.float32
    node = lambda w: pl.BlockSpec((BLK, w), lambda i: (i, 0))
    full = lambda a: pl.BlockSpec(a.shape, lambda i: (0,) * a.ndim)
    wargs = (p['gcn1_b'].reshape(1, 64), p['hgc1_b'].reshape(1, 64),
             p['gcn2_w'], p['gcn2_b'].reshape(1, 64),
             p['hgc2_w'], p['hgc2_b'].reshape(1, 64),
             p['tl_out_w'], p['tl_out_b'].reshape(1, 64),
             p['ctx_w'], p['ctx_b'].reshape(1, 64),
             p['fuse_w'], p['fuse_b'].reshape(1, 128),
             p['bin_w'], p['bin_b'].reshape(1, 1),
             p['c3_w'], p['c3_b'].reshape(1, 3),
             p['type_w'], p['type_b'].reshape(1, 12))
    fn = pl.pallas_call(
        _tcb_body,
        grid=(NBLK,),
        in_specs=[node(32)] * 8 + [node(1)] * 6
                 + [pl.BlockSpec((1, 64), lambda i: (0, 0)),
                    pl.BlockSpec((1, 7), lambda i: (0, 0))]
                 + [full(w) for w in wargs],
        out_specs=[pl.BlockSpec((1, 1), lambda i: (0, 0)),
                   pl.BlockSpec((1, 3), lambda i: (0, 0)),
                   pl.BlockSpec((1, 12), lambda i: (0, 0))],
        out_shape=[jax.ShapeDtypeStruct((1, 1), f32),
                   jax.ShapeDtypeStruct((1, 3), f32),
                   jax.ShapeDtypeStruct((1, 12), f32)],
        scratch_shapes=[pltpu.VMEM((1, 64), f32), pltpu.VMEM((1, 64), f32)],
    )
    return fn(*zs, *aggs, *asigs, r_t, rh_t, shr, sctx, *wargs)


@jax.jit
def kernel(api_id, status_id, node_id, depth, pos, lat, ctx, edge_index, params):
    p = params
    f32, i32 = jnp.float32, jnp.int32

    def pad2d(a):
        return jnp.pad(a.astype(i32), (0, NP - NN)).reshape(NIDR, 128)

    ids5 = (pad2d(api_id), pad2d(status_id), pad2d(node_id),
            pad2d(depth), pad2d(pos))
    nid1d = jnp.pad(node_id.astype(i32), (0, NP - NN), constant_values=1023)
    src2d = jnp.pad(edge_index[0].astype(i32), (0, EP - EE),
                    constant_values=NN).reshape(ER, 128)
    dst2d = jnp.pad(edge_index[1].astype(i32), (0, EP - EE),
                    constant_values=NN).reshape(ER, 128)
    ones_in = jnp.ones((128,), f32)
    zeros1 = jnp.zeros((128,), f32)
    zeros2 = jnp.zeros((128, 32), f32)
    tabs5 = (p['api_emb'], p['status_emb'], p['node_emb'],
             p['depth_emb'], p['pos_emb'])

    *embs, cnt, ploc, locc = _sc1(src2d, dst2d, ids5, nid1d, tabs5,
                                  ones_in, zeros1)
    pfin, degh, _, _ = _sc2(locc, nid1d, ploc, jnp.arange(NP, dtype=i32))

    zlo, zhi, zhlo, zhhi, r_t, rh_t, shr, sctx = _tca(
        embs, lat, ctx, cnt[0].reshape(NP, 1), cnt[1].reshape(NP, 1),
        degh.reshape(NP, 1), p)

    agg_lo, agg_hi, as0, as1 = _sc3(src2d, dst2d, zlo, zhi, r_t.reshape(NP),
                                    zeros2, zeros1)
    agh_lo, agh_hi, ah0, ah1 = _sc4(pfin.reshape(NIDR, 128), zhlo, zhhi,
                                    rh_t.reshape(NP), zeros2, zeros1)

    ob, oc3, oty = _tcb((zlo, zhi, zhlo, zhhi),
                        (agg_lo, agg_hi, agh_lo, agh_hi),
                        (as0.reshape(NP, 1), as1.reshape(NP, 1),
                         ah0.reshape(NP, 1), ah1.reshape(NP, 1)),
                        r_t, rh_t, shr, sctx, p)
    return ob.reshape(1), oc3, oty


# speedup vs baseline: 12.1706x; 1.0988x over previous
import functools
import jax
import jax.numpy as jnp
from jax import lax
from jax.experimental import pallas as pl
from jax.experimental.pallas import tpu as pltpu
from jax.experimental.pallas import tpu_sc as plsc

NN = 50000
NP = 53248
PW = 1664
NB = PW // 128
NIDR = NP // 128
EE = 800000
EP = 802816
ER = EP // 128
ERW = ER // 32
ERT = ER // 16
TILE_N = NP // 16
TRASH = NP
GCF = 64
BLK = 2000
NBLK = NN // BLK


def _dot(a, b):
    return lax.dot_general(a, b, (((1,), (0,)), ((), ())),
                           precision=lax.Precision.HIGHEST,
                           preferred_element_type=jnp.float32)


def _bdot(a, b):
    return lax.dot_general(a.astype(jnp.bfloat16), b.astype(jnp.bfloat16),
                           (((1,), (0,)), ((), ())),
                           preferred_element_type=jnp.float32)


def _wdot(a, b):
    return _dot(a, b.astype(jnp.bfloat16).astype(jnp.float32))


def _mesh():
    return plsc.VectorSubcoreMesh(core_axis_name="c", subcore_axis_name="s")


def _sc1_body(src2d, dst2d, i_api, i_sta, i_nod, i_dep, i_pos, nid1d,
              t_api, t_sta, t_nod, t_dep, t_pos, ones_in, zeros1,
              e_api, e_sta, e_nod, e_dep, e_pos, cnt_out, ploc_out, locc_out,
              idxs, rows_big, onesv, zb1, idsv, lastv, plocv, last_sm, sem,
              cnt_sp):
    c = lax.axis_index("c")
    s = lax.axis_index("s")
    wid = c * 16 + s

    pltpu.sync_copy(zeros1, zb1)

    def zloop(k, _):
        pltpu.sync_copy(zb1, cnt_sp.at[pl.ds(s * TILE_N + k * 128, 128)])
        return 0
    lax.fori_loop(0, TILE_N // 128, zloop, 0)

    base = wid * PW
    for tab, ids, out in ((t_api, i_api, e_api), (t_sta, i_sta, e_sta),
                          (t_nod, i_nod, e_nod), (t_dep, i_dep, e_dep),
                          (t_pos, i_pos, e_pos)):
        descs = []
        for j in range(NB):
            row = wid * NB + j
            pltpu.sync_copy(ids.at[row], idxs[j])
            descs.append(pltpu.async_copy(
                tab.at[idxs[j]], rows_big.at[pl.ds(j * 128, 128)], sem))
        for dsc in descs:
            dsc.wait()
        pltpu.sync_copy(rows_big, out.at[pl.ds(base, PW)])

    pltpu.sync_copy(nid1d.at[pl.ds(base, PW)], idsv)

    def linit(j, _):
        last_sm[j] = jnp.int32(-1)
        return 0
    lax.fori_loop(0, 1024, linit, 0)

    lane_iota = lax.iota(jnp.int32, 16)

    def scan(g, _):
        d = pl.ds(g * 16, 16)
        vv = idsv[d]
        gbase = base + g * 16
        pv = jnp.full((16,), -1, jnp.int32)
        for lane in range(16):
            v = vv[lane]
            prev = last_sm[v]
            last_sm[v] = gbase + lane
            pv = jnp.where(lane_iota == lane, prev, pv)
        plocv[d] = pv
        return 0
    lax.fori_loop(0, PW // 16, scan, 0)
    pltpu.sync_copy(plocv, ploc_out.at[pl.ds(base, PW)])

    def l2v(g, _):
        pv = jnp.full((16,), -1, jnp.int32)
        for lane in range(16):
            pv = jnp.where(lane_iota == lane, last_sm[g * 16 + lane], pv)
        lastv[pl.ds(g * 16, 16)] = pv
        return 0
    lax.fori_loop(0, 64, l2v, 0)
    pltpu.sync_copy(lastv, locc_out.at[wid])

    plsc.subcore_barrier()
    pltpu.sync_copy(ones_in, onesv)

    def cblk(j2, _):
        rowA = wid * ERW + 2 * j2
        rowB = rowA + 1
        pltpu.sync_copy(src2d.at[rowA], idxs[0])
        pltpu.sync_copy(dst2d.at[rowA], idxs[1])
        pltpu.sync_copy(src2d.at[rowB], idxs[2])
        pltpu.sync_copy(dst2d.at[rowB], idxs[3])
        ds_ = [pltpu.async_copy(onesv, cnt_sp.at[idxs[k]], sem, add=True)
               for k in range(4)]
        for dsc in ds_:
            dsc.wait()
        return 0
    lax.fori_loop(0, ERW // 2, cblk, 0)

    plsc.subcore_barrier()
    pltpu.sync_copy(cnt_sp.at[pl.ds(s * TILE_N, TILE_N)],
                    cnt_out.at[c, pl.ds(s * TILE_N, TILE_N)])


def _sc1(src2d, dst2d, ids5, nid1d, tabs5, ones_in, zeros1):
    f32, i32 = jnp.float32, jnp.int32
    out_type = ([jax.ShapeDtypeStruct((NP, 32), f32)] * 5
                + [jax.ShapeDtypeStruct((2, NP), f32),
                   jax.ShapeDtypeStruct((NP,), i32),
                   jax.ShapeDtypeStruct((32, 1024), i32)])
    fn = pl.kernel(
        _sc1_body, out_type=out_type, mesh=_mesh(),
        compiler_params=pltpu.CompilerParams(use_tc_tiling_on_sc=False),
        scratch_types=[
            [pltpu.VMEM((128,), i32) for _ in range(NB)],
            pltpu.VMEM((PW, 32), f32),
            pltpu.VMEM((128,), f32),
            pltpu.VMEM((128,), f32),
            pltpu.VMEM((PW,), i32),
            pltpu.VMEM((1024,), i32),
            pltpu.VMEM((PW,), i32),
            pltpu.SMEM((1024,), i32),
            pltpu.SemaphoreType.DMA,
            pltpu.VMEM_SHARED((NP,), f32),
        ])
    return fn(src2d, dst2d, *ids5, nid1d, *tabs5, ones_in, zeros1)


def _sc2_body(locc, nid1d, ploc, arange1d,
              pfin, degh, car_out, lg_out,
              rowv, carv, lgv, idsv, plv, pfv, dhv, idxb, cvals, lvals, arv, sem):
    c = lax.axis_index("c")
    s = lax.axis_index("s")
    wid = c * 16 + s
    base = wid * PW

    def vinit(j, _):
        carv[pl.ds(j * 16, 16)] = jnp.full((16,), -1, jnp.int32)
        lgv[pl.ds(j * 16, 16)] = jnp.full((16,), -1, jnp.int32)
        return 0
    lax.fori_loop(0, 64, vinit, 0)

    def rloop(j, _):
        pltpu.sync_copy(locc.at[j], rowv)
        use = (j < wid)

        def vmax(g, _):
            d = pl.ds(g * 16, 16)
            rv = rowv[d]
            lgv[d] = jnp.maximum(lgv[d], rv)
            carv[d] = jnp.where(use, jnp.maximum(carv[d], rv), carv[d])
            return 0
        lax.fori_loop(0, 64, vmax, 0)
        return 0
    lax.fori_loop(0, 32, rloop, 0)

    woff = wid * 1024
    pltpu.sync_copy(carv, car_out.at[pl.ds(woff, 1024)])
    pltpu.sync_copy(lgv, lg_out.at[pl.ds(woff, 1024)])
    pltpu.sync_copy(nid1d.at[pl.ds(base, PW)], idsv)
    pltpu.sync_copy(ploc.at[pl.ds(base, PW)], plv)

    pltpu.sync_copy(arange1d.at[pl.ds(base, PW)], arv)
    for j in range(NB):
        def mkidx(g, _, j=j):
            idxb[pl.ds(g * 16, 16)] = idsv[pl.ds(j * 128 + g * 16, 16)] + woff
            return 0
        lax.fori_loop(0, 8, mkidx, 0)
        pltpu.async_copy(car_out.at[idxb], cvals, sem).wait()
        pltpu.async_copy(lg_out.at[idxb], lvals, sem).wait()

        def grp(g, _, j=j):
            d16 = pl.ds(g * 16, 16)
            dall = pl.ds(j * 128 + g * 16, 16)
            pv = plv[dall]
            pf = jnp.where(pv >= 0, pv, cvals[d16])
            pex = jnp.where(pf >= 0, 1, 0)
            nex = jnp.where(lvals[d16] != arv[dall], 1, 0)
            pfv[dall] = pf
            dhv[dall] = (1 + pex + nex).astype(jnp.float32)
            return 0
        lax.fori_loop(0, 8, grp, 0)

    pltpu.sync_copy(pfv, pfin.at[pl.ds(base, PW)])
    pltpu.sync_copy(dhv, degh.at[pl.ds(base, PW)])


def _sc2(locc, nid1d, ploc, arange1d):
    f32, i32 = jnp.float32, jnp.int32
    fn = pl.kernel(
        _sc2_body,
        out_type=[jax.ShapeDtypeStruct((NP,), i32),
                  jax.ShapeDtypeStruct((NP,), f32),
                  jax.ShapeDtypeStruct((32 * 1024,), i32),
                  jax.ShapeDtypeStruct((32 * 1024,), i32)],
        mesh=_mesh(),
        compiler_params=pltpu.CompilerParams(use_tc_tiling_on_sc=False),
        scratch_types=[
            pltpu.VMEM((1024,), i32),
            pltpu.VMEM((1024,), i32),
            pltpu.VMEM((1024,), i32),
            pltpu.VMEM((PW,), i32),
            pltpu.VMEM((PW,), i32),
            pltpu.VMEM((PW,), i32),
            pltpu.VMEM((PW,), f32),
            pltpu.VMEM((128,), i32),
            pltpu.VMEM((128,), i32),
            pltpu.VMEM((128,), i32),
            pltpu.VMEM((PW,), i32),
            pltpu.SemaphoreType.DMA,
        ])
    return fn(locc, nid1d, ploc, arange1d)


def _sc3_body(src2d, dst2d, zlo_t, zhi_t, r_t, zeros2, zeros1,
              agg_lo, agg_hi, asig0, asig1,
              sidxA, didxA, sidxB, didxB, rowsA0, rowsA1, rowsB0, rowsB1,
              rvals0, rvals1, zb1, semZ, agg_sp, asig_sp):
    c = lax.axis_index("c")
    s = lax.axis_index("s")

    pltpu.sync_copy(zeros2, rowsA0)
    pltpu.sync_copy(zeros1, zb1)

    def zloop(k, _):
        pltpu.sync_copy(rowsA0, agg_sp.at[pl.ds(s * TILE_N + k * 128, 128)])
        pltpu.sync_copy(zb1, asig_sp.at[pl.ds(s * TILE_N + k * 128, 128)])
        return 0
    lax.fori_loop(0, TILE_N // 128, zloop, 0)
    plsc.subcore_barrier()

    def edge_pass(ztab, parity):
        def eblk(j2, _):
            rowA = s * ERT + 2 * j2
            rowB = rowA + 1
            pltpu.sync_copy(src2d.at[rowA], sidxA)
            pltpu.sync_copy(dst2d.at[rowA], didxA)
            a0 = pltpu.async_copy(ztab.at[sidxA], rowsA0, semZ)
            a1 = pltpu.async_copy(ztab.at[didxA], rowsA1, semZ)
            pltpu.sync_copy(src2d.at[rowB], sidxB)
            pltpu.sync_copy(dst2d.at[rowB], didxB)
            b0 = pltpu.async_copy(ztab.at[sidxB], rowsB0, semZ)
            b1 = pltpu.async_copy(ztab.at[didxB], rowsB1, semZ)
            if parity == 0:
                r0 = pltpu.async_copy(r_t.at[didxA], rvals0, semZ)
                r1 = pltpu.async_copy(r_t.at[sidxA], rvals1, semZ)
            else:
                r0 = pltpu.async_copy(r_t.at[didxB], rvals0, semZ)
                r1 = pltpu.async_copy(r_t.at[sidxB], rvals1, semZ)
            a0.wait()
            a1.wait()
            b0.wait()
            b1.wait()
            r0.wait()
            r1.wait()
            pltpu.sync_copy(rowsA0, agg_sp.at[didxA], add=True)
            pltpu.sync_copy(rowsA1, agg_sp.at[sidxA], add=True)
            pltpu.sync_copy(rowsB0, agg_sp.at[didxB], add=True)
            pltpu.sync_copy(rowsB1, agg_sp.at[sidxB], add=True)
            if parity == 0:
                pltpu.sync_copy(rvals0, asig_sp.at[sidxA], add=True)
                pltpu.sync_copy(rvals1, asig_sp.at[didxA], add=True)
            else:
                pltpu.sync_copy(rvals0, asig_sp.at[sidxB], add=True)
                pltpu.sync_copy(rvals1, asig_sp.at[didxB], add=True)
            return 0
        lax.fori_loop(0, ERT // 2, eblk, 0)

    @pl.when(c == 0)
    def _():
        edge_pass(zlo_t, 0)

    @pl.when(c == 1)
    def _():
        edge_pass(zhi_t, 1)

    plsc.subcore_barrier()
    d = pl.ds(s * TILE_N, TILE_N)

    @pl.when(c == 0)
    def _():
        pltpu.sync_copy(agg_sp.at[d], agg_lo.at[d])
        pltpu.sync_copy(asig_sp.at[d], asig0.at[d])

    @pl.when(c == 1)
    def _():
        pltpu.sync_copy(agg_sp.at[d], agg_hi.at[d])
        pltpu.sync_copy(asig_sp.at[d], asig1.at[d])


def _sc3(src2d, dst2d, zlo, zhi, r_t, zeros2, zeros1):
    f32, i32 = jnp.float32, jnp.int32
    fn = pl.kernel(
        _sc3_body,
        out_type=[jax.ShapeDtypeStruct((NP, 32), f32),
                  jax.ShapeDtypeStruct((NP, 32), f32),
                  jax.ShapeDtypeStruct((NP,), f32),
                  jax.ShapeDtypeStruct((NP,), f32)],
        mesh=_mesh(),
        compiler_params=pltpu.CompilerParams(use_tc_tiling_on_sc=False),
        scratch_types=[
            pltpu.VMEM((128,), i32),
            pltpu.VMEM((128,), i32),
            pltpu.VMEM((128,), i32),
            pltpu.VMEM((128,), i32),
            pltpu.VMEM((128, 32), f32),
            pltpu.VMEM((128, 32), f32),
            pltpu.VMEM((128, 32), f32),
            pltpu.VMEM((128, 32), f32),
            pltpu.VMEM((128,), f32),
            pltpu.VMEM((128,), f32),
            pltpu.VMEM((128,), f32),
            pltpu.SemaphoreType.DMA,
            pltpu.VMEM_SHARED((NP, 32), f32),
            pltpu.VMEM_SHARED((NP,), f32),
        ])
    return fn(src2d, dst2d, zlo, zhi, r_t, zeros2, zeros1)


def _sc4_body(pf2d, zhlo_t, zhhi_t, rh_t, zeros2, zeros1,
              agh_lo, agh_hi, ash0, ash1,
              pv, gp, si, sp_, rows, rvals, zb2, zb1, sem, agg_sp, asig_sp):
    c = lax.axis_index("c")
    s = lax.axis_index("s")
    wid = c * 16 + s

    pltpu.sync_copy(zeros2, zb2)
    pltpu.sync_copy(zeros1, zb1)

    def zloop(k, _):
        pltpu.sync_copy(zb2, agg_sp.at[pl.ds(s * TILE_N + k * 128, 128)])
        pltpu.sync_copy(zb1, asig_sp.at[pl.ds(s * TILE_N + k * 128, 128)])
        return 0
    lax.fori_loop(0, TILE_N // 128, zloop, 0)
    plsc.subcore_barrier()

    def host_pass(ztab, parity):
        def blk(j, _):
            row = s * (2 * NB) + j
            nodebase = row * 128
            pltpu.sync_copy(pf2d.at[row], pv)

            def grp(g, _):
                d = pl.ds(g * 16, 16)
                pvv = pv[d]
                ivv = nodebase + g * 16 + lax.iota(jnp.int32, 16)
                m = pvv >= 0
                gp[d] = jnp.where(m, pvv, 0)
                si[d] = jnp.where(m, ivv, TRASH)
                sp_[d] = jnp.where(m, pvv, TRASH)
                return 0
            lax.fori_loop(0, 8, grp, 0)

            pltpu.async_copy(ztab.at[gp], rows, sem).wait()
            pltpu.sync_copy(rows, agg_sp.at[si], add=True)
            pltpu.sync_copy(ztab.at[pl.ds(nodebase, 128)], rows)
            pltpu.sync_copy(rows, agg_sp.at[sp_], add=True)

            @pl.when(j % 2 == parity)
            def _():
                pltpu.async_copy(rh_t.at[gp], rvals, sem).wait()
                pltpu.sync_copy(rvals, asig_sp.at[si], add=True)
                pltpu.sync_copy(rh_t.at[pl.ds(nodebase, 128)], rvals)
                pltpu.sync_copy(rvals, asig_sp.at[sp_], add=True)
            return 0
        lax.fori_loop(0, 2 * NB, blk, 0)

    @pl.when(c == 0)
    def _():
        host_pass(zhlo_t, 0)

    @pl.when(c == 1)
    def _():
        host_pass(zhhi_t, 1)

    plsc.subcore_barrier()
    d = pl.ds(s * TILE_N, TILE_N)

    @pl.when(c == 0)
    def _():
        pltpu.sync_copy(agg_sp.at[d], agh_lo.at[d])
        pltpu.sync_copy(asig_sp.at[d], ash0.at[d])

    @pl.when(c == 1)
    def _():
        pltpu.sync_copy(agg_sp.at[d], agh_hi.at[d])
        pltpu.sync_copy(asig_sp.at[d], ash1.at[d])


def _sc4(pf2d, zhlo, zhhi, rh_t, zeros2, zeros1):
    f32, i32 = jnp.float32, jnp.int32
    fn = pl.kernel(
        _sc4_body,
        out_type=[jax.ShapeDtypeStruct((NP, 32), f32),
                  jax.ShapeDtypeStruct((NP, 32), f32),
                  jax.ShapeDtypeStruct((NP,), f32),
                  jax.ShapeDtypeStruct((NP,), f32)],
        mesh=_mesh(),
        compiler_params=pltpu.CompilerParams(use_tc_tiling_on_sc=False),
        scratch_types=[
            pltpu.VMEM((128,), i32),
            pltpu.VMEM((128,), i32),
            pltpu.VMEM((128,), i32),
            pltpu.VMEM((128,), i32),
            pltpu.VMEM((128, 32), f32),
            pltpu.VMEM((128,), f32),
            pltpu.VMEM((128, 32), f32),
            pltpu.VMEM((128,), f32),
            pltpu.SemaphoreType.DMA,
            pltpu.VMEM_SHARED((TRASH + 8, 32), f32),
            pltpu.VMEM_SHARED((TRASH + 8,), f32),
        ])
    return fn(pf2d, zhlo, zhhi, rh_t, zeros2, zeros1)


def _tca_body(eapi, esta, enod, edep, epos, lat, ctx, cnt0, cnt1, degh,
              lw1, lb1, lw2, lb2, mw, mb, wiou, biou,
              zlo, zhi, zhlo, zhhi, r_o, rh_o, shr_o, sctx_o):
    i = pl.program_id(0)
    lt = _bdot(jnp.maximum(_bdot(lat[...], lw1[...]) + lb1[...], 0.0),
               lw2[...]) + lb2[...]
    feat = jnp.concatenate(
        [eapi[...], esta[...], enod[...], edep[...], epos[...], lt], axis=1)
    x = jnp.maximum(_bdot(feat, mw[...]) + mb[...], 0.0)

    r = 1.0 / jnp.sqrt(cnt0[...] + cnt1[...] + 1.0)
    rh = 1.0 / jnp.sqrt(degh[...])
    xn = x * r
    xnh = x * rh
    zlo[...] = xn[:, :32]
    zhi[...] = xn[:, 32:]
    zhlo[...] = xnh[:, :32]
    zhhi[...] = xnh[:, 32:]
    r_o[...] = r
    rh_o[...] = rh

    iou = _bdot(x, wiou[...]) + biou[...]
    ig = jax.nn.sigmoid(iou[:, :GCF])
    og = jax.nn.sigmoid(iou[:, GCF:2 * GCF])
    ug = jnp.tanh(iou[:, 2 * GCF:])
    h = og * jnp.tanh(ig * ug)
    hr = jnp.maximum(h, 0.0)

    @pl.when(i == 0)
    def _():
        shr_o[...] = jnp.zeros_like(shr_o)
        sctx_o[...] = jnp.zeros_like(sctx_o)

    shr_o[...] += jnp.sum(hr, axis=0, keepdims=True)
    sctx_o[...] += jnp.sum(ctx[...], axis=0, keepdims=True)


def _tca(embs, lat, ctx, cnt0, cnt1, degh, p):
    f32 = jnp.float32
    node = lambda w: pl.BlockSpec((BLK, w), lambda i: (i, 0))
    full = lambda a: pl.BlockSpec(a.shape, lambda i: (0,) * a.ndim)
    acc = lambda w: pl.BlockSpec((1, w), lambda i: (0, 0))
    wargs = (p['lat_w1'], p['lat_b1'].reshape(1, 32), p['lat_w2'],
             p['lat_b2'].reshape(1, 32), p['merge_w'],
             p['merge_b'].reshape(1, 64),
             p['W_iouf'][:, :3 * GCF], p['b_iou'])
    fn = pl.pallas_call(
        _tca_body,
        grid=(NBLK,),
        in_specs=[node(32)] * 5 + [node(1), node(7), node(1), node(1), node(1)]
                 + [full(w) for w in wargs],
        out_specs=[node(32)] * 4 + [node(1)] * 2 + [acc(64), acc(7)],
        out_shape=[jax.ShapeDtypeStruct((NP, 32), f32)] * 4
                  + [jax.ShapeDtypeStruct((NP, 1), f32)] * 2
                  + [jax.ShapeDtypeStruct((1, 64), f32),
                     jax.ShapeDtypeStruct((1, 7), f32)],
    )
    return fn(*embs, lat, ctx, cnt0, cnt1, degh, *wargs)


def _tcb_body(zlo, zhi, zhlo, zhhi, alo, ahi, hlo, hhi,
              as0, as1, ah0, ah1, r_i, rh_i, shr, sctx,
              g1w, g1b, h1w, h1b, g2w, g2b, hg2w, hg2b,
              tow, tob, cw, cb, fw, fb, bw, bb, c3w, c3b, tw, tb,
              ob, oc3, oty, acc_c, acc_h):
    i = pl.program_id(0)

    @pl.when(i == 0)
    def _():
        acc_c[...] = jnp.zeros_like(acc_c)
        acc_h[...] = jnp.zeros_like(acc_h)

    r = r_i[...]
    xn = jnp.concatenate([zlo[...], zhi[...]], axis=1)
    agg = jnp.concatenate([alo[...], ahi[...]], axis=1)
    h1 = jnp.maximum(_bdot(r * (agg + xn), g1w[...]) + g1b[...], 0.0)
    alpha = r * (as0[...] + as1[...] + r)
    acc_c[...] += jnp.sum(alpha * h1, axis=0, keepdims=True)

    rh = rh_i[...]
    xnh = jnp.concatenate([zhlo[...], zhhi[...]], axis=1)
    aggh = jnp.concatenate([hlo[...], hhi[...]], axis=1)
    h1h = jnp.maximum(_bdot(rh * (aggh + xnh), h1w[...]) + h1b[...], 0.0)
    alphah = rh * (ah0[...] + ah1[...] + rh)
    acc_h[...] += jnp.sum(alphah * h1h, axis=0, keepdims=True)

    @pl.when(i == NBLK - 1)
    def _():
        inv = 1.0 / NN
        mc = _wdot(acc_c[...] * inv, g2w[...]) + g2b[...]
        mh = _wdot(acc_h[...] * inv, hg2w[...]) + hg2b[...]
        mt = _wdot(shr[...] * inv, tow[...]) + tob[...]
        ch = jnp.maximum(_bdot(sctx[...] * inv, cw[...]) + cb[...], 0.0)
        fused = jnp.maximum(
            _bdot(jnp.concatenate([mc, mh, mt, ch], axis=1), fw[...])
            + fb[...], 0.0)
        ob[...] = _bdot(fused, bw[...]) + bb[...]
        oc3[...] = _bdot(fused, c3w[...]) + c3b[...]
        oty[...] = _bdot(fused, tw[...]) + tb[...]


def _tcb(zs, aggs, asigs, r_t, rh_t, shr, sctx, p):
    f32 = jnp.float32
    node = lambda w: pl.BlockSpec((BLK, w), lambda i: (i, 0))
    full = lambda a: pl.BlockSpec(a.shape, lambda i: (0,) * a.ndim)
    wargs = (p['gcn1_w'], p['gcn1_b'].reshape(1, 64),
             p['hgc1_w'], p['hgc1_b'].reshape(1, 64),
             p['gcn2_w'], p['gcn2_b'].reshape(1, 64),
             p['hgc2_w'], p['hgc2_b'].reshape(1, 64),
             p['tl_out_w'], p['tl_out_b'].reshape(1, 64),
             p['ctx_w'], p['ctx_b'].reshape(1, 64),
             p['fuse_w'], p['fuse_b'].reshape(1, 128),
             p['bin_w'], p['bin_b'].reshape(1, 1),
             p['c3_w'], p['c3_b'].reshape(1, 3),
             p['type_w'], p['type_b'].reshape(1, 12))
    fn = pl.pallas_call(
        _tcb_body,
        grid=(NBLK,),
        in_specs=[node(32)] * 8 + [node(1)] * 6
                 + [pl.BlockSpec((1, 64), lambda i: (0, 0)),
                    pl.BlockSpec((1, 7), lambda i: (0, 0))]
                 + [full(w) for w in wargs],
        out_specs=[pl.BlockSpec((1, 1), lambda i: (0, 0)),
                   pl.BlockSpec((1, 3), lambda i: (0, 0)),
                   pl.BlockSpec((1, 12), lambda i: (0, 0))],
        out_shape=[jax.ShapeDtypeStruct((1, 1), f32),
                   jax.ShapeDtypeStruct((1, 3), f32),
                   jax.ShapeDtypeStruct((1, 12), f32)],
        scratch_shapes=[pltpu.VMEM((1, 64), f32), pltpu.VMEM((1, 64), f32)],
    )
    return fn(*zs, *aggs, *asigs, r_t, rh_t, shr, sctx, *wargs)


@jax.jit
def kernel(api_id, status_id, node_id, depth, pos, lat, ctx, edge_index, params):
    p = params
    f32, i32 = jnp.float32, jnp.int32

    def pad2d(a):
        return jnp.pad(a.astype(i32), (0, NP - NN)).reshape(NIDR, 128)

    ids5 = (pad2d(api_id), pad2d(status_id), pad2d(node_id),
            pad2d(depth), pad2d(pos))
    nid1d = jnp.pad(node_id.astype(i32), (0, NP - NN), constant_values=1023)
    src2d = jnp.pad(edge_index[0].astype(i32), (0, EP - EE),
                    constant_values=NN).reshape(ER, 128)
    dst2d = jnp.pad(edge_index[1].astype(i32), (0, EP - EE),
                    constant_values=NN).reshape(ER, 128)
    ones_in = jnp.ones((128,), f32)
    zeros1 = jnp.zeros((128,), f32)
    zeros2 = jnp.zeros((128, 32), f32)
    tabs5 = (p['api_emb'], p['status_emb'], p['node_emb'],
             p['depth_emb'], p['pos_emb'])

    *embs, cnt, ploc, locc = _sc1(src2d, dst2d, ids5, nid1d, tabs5,
                                  ones_in, zeros1)
    pfin, degh, _, _ = _sc2(locc, nid1d, ploc, jnp.arange(NP, dtype=i32))

    zlo, zhi, zhlo, zhhi, r_t, rh_t, shr, sctx = _tca(
        embs, lat, ctx, cnt[0].reshape(NP, 1), cnt[1].reshape(NP, 1),
        degh.reshape(NP, 1), p)

    agg_lo, agg_hi, as0, as1 = _sc3(src2d, dst2d, zlo, zhi, r_t.reshape(NP),
                                    zeros2, zeros1)
    agh_lo, agh_hi, ah0, ah1 = _sc4(pfin.reshape(NIDR, 128), zhlo, zhhi,
                                    rh_t.reshape(NP), zeros2, zeros1)

    ob, oc3, oty = _tcb((zlo, zhi, zhlo, zhhi),
                        (agg_lo, agg_hi, agh_lo, agh_hi),
                        (as0.reshape(NP, 1), as1.reshape(NP, 1),
                         ah0.reshape(NP, 1), ah1.reshape(NP, 1)),
                        r_t, rh_t, shr, sctx, p)
    return ob.reshape(1), oc3, oty
